# Initial kernel scaffold; baseline (speedup 1.0000x reference)
import jax
import jax.numpy as jnp
from jax.experimental import pallas as pl


def kernel(e, x, t, r_ij, neighbors, neighbor_mask, f_ij, W_filter, b_filter, Wq, Wk, Wv, Wo):
    raise NotImplementedError("write your pallas kernel here")


# speedup vs baseline: 3.3791x; 3.3791x over previous
import functools

import jax
import jax.numpy as jnp
from jax import lax
from jax.experimental import pallas as pl
from jax.experimental.pallas import tpu as pltpu
from jax.experimental.pallas import tpu_sc as plsc

_CUTOFF = 5.0
_NUM_HEADS = 8

_NC = 2
_NS = 16
_NW = _NC * _NS


def _prep_body(x_ref, e_ref, t_ref, h_ref):
    h_ref[...] = x_ref[...] + e_ref[...] + t_ref[...]


def _compute_h(x2, e2, t2):
    a, f = x2.shape
    ba = 1000
    grid = a // ba
    spec = pl.BlockSpec((ba, f), lambda i: (i, 0))
    return pl.pallas_call(
        _prep_body,
        grid=(grid,),
        in_specs=[spec, spec, spec],
        out_specs=spec,
        out_shape=jax.ShapeDtypeStruct((a, f), jnp.float32),
    )(x2, e2, t2)


def _sc_gather(table, idx3, n_rows, d, n_ch, ch):
    mesh = plsc.VectorSubcoreMesh(core_axis_name="c", subcore_axis_name="s")
    e_per_w = n_ch * ch

    @functools.partial(
        pl.kernel,
        mesh=mesh,
        out_type=jax.ShapeDtypeStruct((n_rows, d), jnp.float32),
        scratch_types=[
            pltpu.VMEM((n_ch, ch), jnp.int32),
            pltpu.VMEM((ch, d), jnp.float32),
            pltpu.SemaphoreType.DMA,
        ],
    )
    def gather_kernel(table_hbm, idx_hbm, out_hbm, idx_v, rows_v, sem):
        cid = lax.axis_index("c")
        sid = lax.axis_index("s")
        wid = sid * _NC + cid
        base = wid * e_per_w
        pltpu.sync_copy(idx_hbm.at[wid], idx_v)

        def body(c, carry):
            pltpu.async_copy(table_hbm.at[idx_v.at[c]], rows_v, sem).wait()
            pltpu.sync_copy(rows_v, out_hbm.at[pl.ds(base + c * ch, ch)])
            return carry

        lax.fori_loop(0, n_ch, body, 0)

    return gather_kernel(table, idx3)


def _fused_body(x_ref, h_ref, r_ref, mask_ref, fij_ref, nbh_ref,
                wf_ref, bf_ref, wq_ref, wk_ref, wv_ref, wo_ref, out_ref,
                *, ba, nbh, f, heads):
    dh = f // heads
    rows = ba * nbh

    wfilt = jnp.dot(fij_ref[...], wf_ref[...],
                    preferred_element_type=jnp.float32) + bf_ref[...]

    r = r_ref[...]
    c = 0.5 * (jnp.cos(jnp.pi * r / _CUTOFF) + 1.0)
    c = jnp.where(r < _CUTOFF, c, 0.0) * mask_ref[...]

    m3 = (nbh_ref[...].reshape(ba, nbh, f)
          * wfilt.reshape(ba, nbh, f)
          * c[:, :, None])
    m = m3.reshape(rows, f)

    q = jnp.dot(h_ref[...], wq_ref[...], preferred_element_type=jnp.float32)
    k = jnp.dot(m, wk_ref[...], preferred_element_type=jnp.float32)
    v = jnp.dot(m, wv_ref[...], preferred_element_type=jnp.float32)

    di = lax.broadcasted_iota(jnp.int32, (f, heads), 0)
    hi = lax.broadcasted_iota(jnp.int32, (f, heads), 1)
    seg = (di // dh == hi).astype(jnp.float32)

    qr = jnp.broadcast_to(q.reshape(ba, 1, f), (ba, nbh, f)).reshape(rows, f)
    prod = qr * k
    logits = jnp.dot(prod, seg, preferred_element_type=jnp.float32)
    logits = logits * (1.0 / (dh ** 0.5))

    lg3 = logits.reshape(ba, nbh, heads)
    lg3 = jnp.where(mask_ref[...][:, :, None] > 0, lg3, -1e9)
    mx = jnp.max(lg3, axis=1, keepdims=True)
    p = jnp.exp(lg3 - mx)
    s = jnp.sum(p, axis=1, keepdims=True)
    attn = (p / s).reshape(rows, heads)

    attn_f = jnp.dot(attn, seg.T, preferred_element_type=jnp.float32)
    agg = jnp.sum((attn_f * v).reshape(ba, nbh, f), axis=1)

    out = jnp.dot(agg, wo_ref[...], preferred_element_type=jnp.float32)
    out_ref[...] = x_ref[...] + out


def _fused(x2, h2, r2, mask2, fij2, nbh2, wf, bf, wq, wk, wv, wo):
    a, f = x2.shape
    nbh = r2.shape[1]
    g = wf.shape[0]
    ba = 200
    grid = a // ba

    def rowspec(cols):
        return pl.BlockSpec((ba, cols), lambda i: (i, 0))

    def edgespec(cols):
        return pl.BlockSpec((ba * nbh, cols), lambda i: (i, 0))

    def wspec(r_, c_):
        return pl.BlockSpec((r_, c_), lambda i: (0, 0))

    body = functools.partial(_fused_body, ba=ba, nbh=nbh, f=f,
                             heads=_NUM_HEADS)
    return pl.pallas_call(
        body,
        grid=(grid,),
        in_specs=[
            rowspec(f),
            rowspec(f),
            rowspec(nbh),
            rowspec(nbh),
            edgespec(g),
            edgespec(f),
            wspec(g, f),
            wspec(1, f),
            wspec(f, f),
            wspec(f, f),
            wspec(f, f),
            wspec(f, f),
        ],
        out_specs=rowspec(f),
        out_shape=jax.ShapeDtypeStruct((a, f), jnp.float32),
    )(x2, h2, r2, mask2, fij2, nbh2, wf, bf, wq, wk, wv, wo)


def kernel(e, x, t, r_ij, neighbors, neighbor_mask, f_ij,
           W_filter, b_filter, Wq, Wk, Wv, Wo):
    b, a, nbh = neighbors.shape
    f = x.shape[-1]
    g = f_ij.shape[-1]
    n_rows = b * a * nbh

    x2 = x.reshape(a, f)
    h2 = _compute_h(x2, e.reshape(a, f), t.reshape(a, f))

    ch = 80
    e_per_w = n_rows // _NW
    n_ch = e_per_w // ch
    idx3 = neighbors.reshape(_NW, n_ch, ch).astype(jnp.int32)
    nbh2 = _sc_gather(h2, idx3, n_rows, f, n_ch, ch)

    out2 = _fused(
        x2, h2,
        r_ij.reshape(a, nbh), neighbor_mask.reshape(a, nbh),
        f_ij.reshape(a * nbh, g), nbh2,
        W_filter, b_filter.reshape(1, f), Wq, Wk, Wv, Wo,
    )
    return out2.reshape(b, a, f)


# speedup vs baseline: 3.5594x; 1.0534x over previous
import functools

import jax
import jax.numpy as jnp
from jax import lax
from jax.experimental import pallas as pl
from jax.experimental.pallas import tpu as pltpu
from jax.experimental.pallas import tpu_sc as plsc

_CUTOFF = 5.0
_NUM_HEADS = 8

_NC = 2
_NS = 16
_NW = _NC * _NS


def _prep_body(x_ref, e_ref, t_ref, h_ref):
    h_ref[...] = x_ref[...] + e_ref[...] + t_ref[...]


def _compute_h(x2, e2, t2):
    a, f = x2.shape
    ba = 1000
    grid = a // ba
    spec = pl.BlockSpec((ba, f), lambda i: (i, 0))
    return pl.pallas_call(
        _prep_body,
        grid=(grid,),
        in_specs=[spec, spec, spec],
        out_specs=spec,
        out_shape=jax.ShapeDtypeStruct((a, f), jnp.float32),
    )(x2, e2, t2)


def _sc_gather(table, idx3, n_rows, d, n_ch, ch):
    mesh = plsc.VectorSubcoreMesh(core_axis_name="c", subcore_axis_name="s")
    e_per_w = n_ch * ch

    @functools.partial(
        pl.kernel,
        mesh=mesh,
        out_type=jax.ShapeDtypeStruct((n_rows, d), jnp.float32),
        scratch_types=[
            pltpu.VMEM((n_ch, ch), jnp.int32),
            pltpu.VMEM((ch, d), jnp.float32),
            pltpu.VMEM((ch, d), jnp.float32),
            pltpu.SemaphoreType.DMA,
            pltpu.SemaphoreType.DMA,
            pltpu.SemaphoreType.DMA,
            pltpu.SemaphoreType.DMA,
        ],
    )
    def gather_kernel(table_hbm, idx_hbm, out_hbm, idx_v, buf0, buf1,
                      gsem0, gsem1, wsem0, wsem1):
        cid = lax.axis_index("c")
        sid = lax.axis_index("s")
        wid = sid * _NC + cid
        base = wid * e_per_w
        pltpu.sync_copy(idx_hbm.at[wid], idx_v)

        def gath(c, buf, sem):
            return pltpu.make_async_copy(table_hbm.at[idx_v.at[c]], buf, sem)

        def wrt(c, buf, sem):
            return pltpu.make_async_copy(
                buf, out_hbm.at[pl.ds(base + c * ch, ch)], sem)

        n_pairs = n_ch // 2

        gath(0, buf0, gsem0).start()
        gath(1, buf1, gsem1).start()

        def body(c2, carry):
            a = 2 * c2
            b = a + 1
            gath(a, buf0, gsem0).wait()
            wrt(a, buf0, wsem0).start()
            gath(b, buf1, gsem1).wait()
            wrt(b, buf1, wsem1).start()

            @pl.when(c2 + 1 < n_pairs)
            def _():
                wrt(a, buf0, wsem0).wait()
                gath(a + 2, buf0, gsem0).start()
                wrt(b, buf1, wsem1).wait()
                gath(b + 2, buf1, gsem1).start()

            @pl.when(c2 + 1 == n_pairs)
            def _():
                wrt(a, buf0, wsem0).wait()
                wrt(b, buf1, wsem1).wait()

            return carry

        lax.fori_loop(0, n_pairs, body, 0)

    return gather_kernel(table, idx3)


def _fused_body(x_ref, h_ref, r_ref, mask_ref, fij_ref, nbh_ref,
                wf_ref, bf_ref, wq_ref, wk_ref, wv_ref, wo_ref, out_ref,
                *, ba, nbh, f, heads):
    dh = f // heads
    rows = ba * nbh

    wfilt = jnp.dot(fij_ref[...], wf_ref[...],
                    preferred_element_type=jnp.float32) + bf_ref[...]

    r = r_ref[...]
    c = 0.5 * (jnp.cos(jnp.pi * r / _CUTOFF) + 1.0)
    c = jnp.where(r < _CUTOFF, c, 0.0) * mask_ref[...]

    m3 = (nbh_ref[...].reshape(ba, nbh, f)
          * wfilt.reshape(ba, nbh, f)
          * c[:, :, None])
    m = m3.reshape(rows, f)

    q = jnp.dot(h_ref[...], wq_ref[...], preferred_element_type=jnp.float32)
    k = jnp.dot(m, wk_ref[...], preferred_element_type=jnp.float32)
    v = jnp.dot(m, wv_ref[...], preferred_element_type=jnp.float32)

    di = lax.broadcasted_iota(jnp.int32, (f, heads), 0)
    hi = lax.broadcasted_iota(jnp.int32, (f, heads), 1)
    seg = (di // dh == hi).astype(jnp.float32)

    qr = jnp.broadcast_to(q.reshape(ba, 1, f), (ba, nbh, f)).reshape(rows, f)
    prod = qr * k
    logits = jnp.dot(prod, seg, preferred_element_type=jnp.float32)
    logits = logits * (1.0 / (dh ** 0.5))

    lg3 = logits.reshape(ba, nbh, heads)
    lg3 = jnp.where(mask_ref[...][:, :, None] > 0, lg3, -1e9)
    mx = jnp.max(lg3, axis=1, keepdims=True)
    p = jnp.exp(lg3 - mx)
    s = jnp.sum(p, axis=1, keepdims=True)
    attn = (p / s).reshape(rows, heads)

    attn_f = jnp.dot(attn, seg.T, preferred_element_type=jnp.float32)
    agg = jnp.sum((attn_f * v).reshape(ba, nbh, f), axis=1)

    out = jnp.dot(agg, wo_ref[...], preferred_element_type=jnp.float32)
    out_ref[...] = x_ref[...] + out


def _fused(x2, h2, r2, mask2, fij2, nbh2, wf, bf, wq, wk, wv, wo):
    a, f = x2.shape
    nbh = r2.shape[1]
    g = wf.shape[0]
    ba = 200
    grid = a // ba

    def rowspec(cols):
        return pl.BlockSpec((ba, cols), lambda i: (i, 0))

    def edgespec(cols):
        return pl.BlockSpec((ba * nbh, cols), lambda i: (i, 0))

    def wspec(r_, c_):
        return pl.BlockSpec((r_, c_), lambda i: (0, 0))

    body = functools.partial(_fused_body, ba=ba, nbh=nbh, f=f,
                             heads=_NUM_HEADS)
    return pl.pallas_call(
        body,
        grid=(grid,),
        in_specs=[
            rowspec(f),
            rowspec(f),
            rowspec(nbh),
            rowspec(nbh),
            edgespec(g),
            edgespec(f),
            wspec(g, f),
            wspec(1, f),
            wspec(f, f),
            wspec(f, f),
            wspec(f, f),
            wspec(f, f),
        ],
        out_specs=rowspec(f),
        out_shape=jax.ShapeDtypeStruct((a, f), jnp.float32),
    )(x2, h2, r2, mask2, fij2, nbh2, wf, bf, wq, wk, wv, wo)


def kernel(e, x, t, r_ij, neighbors, neighbor_mask, f_ij,
           W_filter, b_filter, Wq, Wk, Wv, Wo):
    b, a, nbh = neighbors.shape
    f = x.shape[-1]
    g = f_ij.shape[-1]
    n_rows = b * a * nbh

    x2 = x.reshape(a, f)
    h2 = _compute_h(x2, e.reshape(a, f), t.reshape(a, f))

    ch = 40
    e_per_w = n_rows // _NW
    n_ch = e_per_w // ch
    idx3 = neighbors.reshape(_NW, n_ch, ch).astype(jnp.int32)
    nbh2 = _sc_gather(h2, idx3, n_rows, f, n_ch, ch)

    out2 = _fused(
        x2, h2,
        r_ij.reshape(a, nbh), neighbor_mask.reshape(a, nbh),
        f_ij.reshape(a * nbh, g), nbh2,
        W_filter, b_filter.reshape(1, f), Wq, Wk, Wv, Wo,
    )
    return out2.reshape(b, a, f)


# speedup vs baseline: 4.1985x; 1.1795x over previous
import functools

import jax
import jax.numpy as jnp
from jax import lax
from jax.experimental import pallas as pl
from jax.experimental.pallas import tpu as pltpu
from jax.experimental.pallas import tpu_sc as plsc

_CUTOFF = 5.0
_NUM_HEADS = 8

_NC = 2
_NS = 16
_NW = _NC * _NS


def _prep_body(x_ref, e_ref, t_ref, h_ref):
    h_ref[...] = x_ref[...] + e_ref[...] + t_ref[...]


def _compute_h(x2, e2, t2):
    a, f = x2.shape
    ba = 1000
    grid = a // ba
    spec = pl.BlockSpec((ba, f), lambda i: (i, 0))
    return pl.pallas_call(
        _prep_body,
        grid=(grid,),
        in_specs=[spec, spec, spec],
        out_specs=spec,
        out_shape=jax.ShapeDtypeStruct((a, f), jnp.float32),
    )(x2, e2, t2)


def _sc_gather(table, idx2, n_rows, d):
    n_g_pad, ch = idx2.shape
    n_g = n_rows // ch
    base_ch = n_g // _NW
    extra = n_g - base_ch * _NW
    smax = -(-(base_ch + 1 + 7) // 8) * 8 + 8
    assert (n_g_pad - smax) % 8 == 0
    R, K = 6, 3
    mesh = plsc.VectorSubcoreMesh(core_axis_name="c", subcore_axis_name="s")

    @functools.partial(
        pl.kernel,
        mesh=mesh,
        out_type=jax.ShapeDtypeStruct((n_rows, d), jnp.float32),
        scratch_types=[
            pltpu.VMEM((smax, ch), jnp.int32),
            pltpu.VMEM((R * ch, d), jnp.float32),
            pltpu.SemaphoreType.DMA,
            pltpu.SemaphoreType.DMA,
        ],
    )
    def gather_kernel(table_hbm, idx_hbm, out_hbm, idx_v, buf, gsem, wsem):
        cid = lax.axis_index("c")
        sid = lax.axis_index("s")
        wid = sid * _NC + cid
        nch_w = jnp.where(wid < extra, base_ch + 1, base_ch)
        row0 = base_ch * wid + jnp.minimum(wid, extra)
        stage0 = jnp.minimum((row0 // 8) * 8, n_g_pad - smax)
        delta = row0 - stage0
        pltpu.sync_copy(idx_hbm.at[pl.ds(stage0, smax)], idx_v)

        def slot(c):
            return buf.at[pl.ds(lax.rem(c, R) * ch, ch)]

        def gath(c):
            return pltpu.make_async_copy(
                table_hbm.at[idx_v.at[delta + c]], slot(c), gsem)

        def wrt(c):
            return pltpu.make_async_copy(
                slot(c), out_hbm.at[pl.ds((row0 + c) * ch, ch)], wsem)

        for c in range(K):
            gath(c).start()

        def body(c, carry):
            gath(c).wait()
            wrt(c).start()

            @pl.when(c >= R - K)
            def _():
                wrt(c - (R - K)).wait()

            @pl.when(c + K < nch_w)
            def _():
                gath(c + K).start()

            return carry

        lax.fori_loop(0, nch_w, body, 0)

        for i in range(R - K):
            wrt(nch_w - (R - K) + i).wait()

    return gather_kernel(table, idx2)


def _fused_body(x_ref, h_ref, r_ref, mask_ref, fij_ref, nbh_ref,
                wf_ref, bf_ref, wq_ref, wk_ref, wv_ref, wo_ref, out_ref,
                *, ba, nbh, f, heads):
    dh = f // heads
    rows = ba * nbh

    wfilt = jnp.dot(fij_ref[...], wf_ref[...],
                    preferred_element_type=jnp.float32) + bf_ref[...]

    r = r_ref[...]
    c = 0.5 * (jnp.cos(jnp.pi * r / _CUTOFF) + 1.0)
    c = jnp.where(r < _CUTOFF, c, 0.0) * mask_ref[...]

    m3 = (nbh_ref[...].reshape(ba, nbh, f)
          * wfilt.reshape(ba, nbh, f)
          * c[:, :, None])
    m = m3.reshape(rows, f)

    q = jnp.dot(h_ref[...], wq_ref[...], preferred_element_type=jnp.float32)
    k = jnp.dot(m, wk_ref[...], preferred_element_type=jnp.float32)
    v = jnp.dot(m, wv_ref[...], preferred_element_type=jnp.float32)

    di = lax.broadcasted_iota(jnp.int32, (f, heads), 0)
    hi = lax.broadcasted_iota(jnp.int32, (f, heads), 1)
    seg = (di // dh == hi).astype(jnp.float32)

    qr = jnp.broadcast_to(q.reshape(ba, 1, f), (ba, nbh, f)).reshape(rows, f)
    prod = qr * k
    logits = jnp.dot(prod, seg, preferred_element_type=jnp.float32)
    logits = logits * (1.0 / (dh ** 0.5))

    lg3 = logits.reshape(ba, nbh, heads)
    lg3 = jnp.where(mask_ref[...][:, :, None] > 0, lg3, -1e9)
    mx = jnp.max(lg3, axis=1, keepdims=True)
    p = jnp.exp(lg3 - mx)
    s = jnp.sum(p, axis=1, keepdims=True)
    attn = (p / s).reshape(rows, heads)

    attn_f = jnp.dot(attn, seg.T, preferred_element_type=jnp.float32)
    agg = jnp.sum((attn_f * v).reshape(ba, nbh, f), axis=1)

    out = jnp.dot(agg, wo_ref[...], preferred_element_type=jnp.float32)
    out_ref[...] = x_ref[...] + out


def _fused(x2, h2, r2, mask2, fij2, nbh2, wf, bf, wq, wk, wv, wo):
    a, f = x2.shape
    nbh = r2.shape[1]
    g = wf.shape[0]
    ba = 200
    grid = a // ba

    def rowspec(cols):
        return pl.BlockSpec((ba, cols), lambda i: (i, 0))

    def edgespec(cols):
        return pl.BlockSpec((ba * nbh, cols), lambda i: (i, 0))

    def wspec(r_, c_):
        return pl.BlockSpec((r_, c_), lambda i: (0, 0))

    body = functools.partial(_fused_body, ba=ba, nbh=nbh, f=f,
                             heads=_NUM_HEADS)
    return pl.pallas_call(
        body,
        grid=(grid,),
        in_specs=[
            rowspec(f),
            rowspec(f),
            rowspec(nbh),
            rowspec(nbh),
            edgespec(g),
            edgespec(f),
            wspec(g, f),
            wspec(1, f),
            wspec(f, f),
            wspec(f, f),
            wspec(f, f),
            wspec(f, f),
        ],
        out_specs=rowspec(f),
        out_shape=jax.ShapeDtypeStruct((a, f), jnp.float32),
    )(x2, h2, r2, mask2, fij2, nbh2, wf, bf, wq, wk, wv, wo)


def kernel(e, x, t, r_ij, neighbors, neighbor_mask, f_ij,
           W_filter, b_filter, Wq, Wk, Wv, Wo):
    b, a, nbh = neighbors.shape
    f = x.shape[-1]
    g = f_ij.shape[-1]
    n_rows = b * a * nbh

    x2 = x.reshape(a, f)
    h2 = _compute_h(x2, e.reshape(a, f), t.reshape(a, f))

    n_g = n_rows // 128
    n_g_pad = -(-n_g // 8) * 8
    idx2 = neighbors.astype(jnp.int32).reshape(n_g, 128)
    idx2 = jnp.concatenate(
        [idx2, jnp.zeros((n_g_pad - n_g, 128), jnp.int32)], axis=0)
    nbh2 = _sc_gather(h2, idx2, n_rows, f)

    out2 = _fused(
        x2, h2,
        r_ij.reshape(a, nbh), neighbor_mask.reshape(a, nbh),
        f_ij.reshape(a * nbh, g), nbh2,
        W_filter, b_filter.reshape(1, f), Wq, Wk, Wv, Wo,
    )
    return out2.reshape(b, a, f)
